# baseline (speedup 1.0000x reference)
import dataclasses
import functools

import jax
import jax.numpy as jnp
from jax import lax
from jax.experimental import pallas as pl
from jax.experimental.pallas import tpu as pltpu
from jax.experimental.pallas import tpu_sc as plsc

F = 26
V = 100000
E = 16
T = 16
B = 4096
P = F * (F - 1) // 2
G = 16
NCHUNK = (P + G - 1) // G
PPAD = NCHUNK * G
LPAD = (PPAD - P) * E

BT = 256

BF = B * F
NC, NS, L = 2, 16, 16
NW = NC * NS
BPW = BF // NW
NLIN = F * (V + 1)
LV_ROWS = NLIN // L
TAIL0 = NLIN - L


def _sc_gather(fm_table, lin_table, idx_flat):
    mesh = plsc.VectorSubcoreMesh(core_axis_name="c", subcore_axis_name="s")
    cp = pltpu.CompilerParams(
        needs_layout_passes=False, use_tc_tiling_on_sc=False)

    @functools.partial(
        pl.kernel,
        mesh=mesh,
        compiler_params=cp,
        out_type=(
            jax.ShapeDtypeStruct((BF, E), jnp.float32),
            jax.ShapeDtypeStruct((BF, 1), jnp.float32),
        ),
        scratch_types=[
            pltpu.VMEM((BPW,), jnp.int32),
            pltpu.VMEM((BPW, E), jnp.float32),
            pltpu.VMEM((BPW, 1), jnp.float32),
            pltpu.SemaphoreType.DMA,
            pltpu.SemaphoreType.DMA,
        ],
    )
    def k(fm_hbm, lt_hbm, idx_hbm, fm_out, lin_out, idx_v, buf_v, lin_v,
          sem, sem2):
        wid = lax.axis_index("s") * NC + lax.axis_index("c")
        base = wid * BPW
        pltpu.sync_copy(idx_hbm.at[pl.ds(base, BPW)], idx_v)
        cp = pltpu.async_copy(fm_hbm.at[idx_v], buf_v, sem)
        cp2 = pltpu.async_copy(lt_hbm.at[idx_v], lin_v, sem2)
        cp.wait()
        pltpu.sync_copy(buf_v, fm_out.at[pl.ds(base, BPW)])
        cp2.wait()
        pltpu.sync_copy(lin_v, lin_out.at[pl.ds(base, BPW)])

    return k(fm_table, lin_table, idx_flat)


def _afm_body(fm_ref, lin_ref, wbig_ref, hbig_ref, qbig_ref, bias_ref,
              out_ref):
    bt = fm_ref.shape[0]
    fm = fm_ref[...]
    wbig = wbig_ref[...]
    hbig = hbig_ref[...]
    qbig = qbig_ref[...]
    bias = bias_ref[...]

    reps = [jnp.tile(fm[:, i * E:(i + 1) * E], (1, F - 1 - i))
            for i in range(F - 1)]
    rest = [fm[:, (i + 1) * E:] for i in range(F - 1)]
    prod = jnp.concatenate(reps, axis=1) * jnp.concatenate(rest, axis=1)
    prod = jnp.concatenate(
        [prod, jnp.zeros((bt, LPAD), jnp.float32)], axis=1)

    s_parts, q_parts = [], []
    for c in range(NCHUNK):
        pc = prod[:, c * G * E:(c + 1) * G * E]
        zc = jnp.dot(pc, wbig, preferred_element_type=jnp.float32)
        ac = jax.nn.relu(zc + bias)
        s_parts.append(jnp.dot(ac, hbig, preferred_element_type=jnp.float32))
        q_parts.append(jnp.dot(pc, qbig, preferred_element_type=jnp.float32))
    s = jnp.concatenate(s_parts, axis=1)
    q = jnp.concatenate(q_parts, axis=1)

    p_id = jax.lax.broadcasted_iota(jnp.int32, (bt, PPAD), 1)
    mask = p_id < P
    s = jnp.where(mask, s, -1e30)
    m = jnp.max(s, axis=1, keepdims=True)
    e = jnp.where(mask, jnp.exp(s - m), 0.0)
    denom = jnp.sum(e, axis=1, keepdims=True)
    num = jnp.sum(e * q, axis=1, keepdims=True)
    afm = num / denom
    lin_sum = jnp.sum(lin_ref[...], axis=1, keepdims=True)
    out_ref[...] = jax.nn.sigmoid(afm + lin_sum)


def _afm_stage(fm_flat, lin, wbig, hbig, qbig, bias):
    return pl.pallas_call(
        _afm_body,
        grid=(B // BT,),
        in_specs=[
            pl.BlockSpec((BT, F * E), lambda i: (i, 0)),
            pl.BlockSpec((BT, F), lambda i: (i, 0)),
            pl.BlockSpec((G * E, G * 32), lambda i: (0, 0)),
            pl.BlockSpec((G * 32, G), lambda i: (0, 0)),
            pl.BlockSpec((G * E, G), lambda i: (0, 0)),
            pl.BlockSpec((1, G * 32), lambda i: (0, 0)),
        ],
        out_specs=pl.BlockSpec((BT, 1), lambda i: (i, 0)),
        out_shape=jax.ShapeDtypeStruct((B, 1), jnp.float32),
    )(fm_flat, lin, wbig, hbig, qbig, bias)


def kernel(indices, fm_table, linear_table, att_W, att_b, att_h, proj_p,
           training):
    del training
    offsets = (jnp.arange(F, dtype=indices.dtype) * (V + 1))[None, :]
    idx = (indices + offsets).astype(jnp.int32)

    fm_rows, lin_vals = _sc_gather(fm_table, linear_table, idx.reshape(-1))

    wc = jnp.zeros((E, 32), jnp.float32).at[:, :T].set(att_W)
    eye = jnp.eye(G, dtype=jnp.float32)
    wbig = jnp.kron(eye, wc)
    hpad = jnp.zeros((32, 1), jnp.float32).at[:T, 0].set(att_h[:, 0])
    hbig = jnp.kron(eye, hpad)
    qbig = jnp.kron(eye, proj_p)
    bias = jnp.tile(
        jnp.concatenate([att_b, jnp.zeros((T,), jnp.float32)])[None, :],
        (1, G))

    fm_flat = fm_rows.reshape(B, F * E)
    lin = lin_vals.reshape(B, F)
    return _afm_stage(fm_flat, lin, wbig, hbig, qbig, bias)


# speedup vs baseline: 4.1498x; 4.1498x over previous
import dataclasses
import functools

import jax
import jax.numpy as jnp
from jax import lax
from jax.experimental import pallas as pl
from jax.experimental.pallas import tpu as pltpu
from jax.experimental.pallas import tpu_sc as plsc

F = 26
V = 100000
E = 16
T = 16
B = 4096
P = F * (F - 1) // 2
G = 16
NCHUNK = (P + G - 1) // G
PPAD = NCHUNK * G
LPAD = (PPAD - P) * E

BT = 256

BF = B * F
NC, NS, L = 2, 16, 16
NW = NC * NS
BPW = BF // NW
NLIN = F * (V + 1)
LV_ROWS = NLIN // L
TAIL0 = NLIN - L


def _sc_gather(fm_table, lin_view, lin_tail, idx_flat):
    mesh = plsc.VectorSubcoreMesh(core_axis_name="c", subcore_axis_name="s")
    cp = pltpu.CompilerParams(
        needs_layout_passes=False, use_tc_tiling_on_sc=False)

    @functools.partial(
        pl.kernel,
        mesh=mesh,
        compiler_params=cp,
        out_type=(
            jax.ShapeDtypeStruct((BF, E), jnp.float32),
            jax.ShapeDtypeStruct((BF,), jnp.float32),
        ),
        scratch_types=[
            pltpu.VMEM((BPW,), jnp.int32),
            pltpu.VMEM((BPW, E), jnp.float32),
            pltpu.VMEM((BPW,), jnp.int32),
            pltpu.VMEM((BPW,), jnp.float32),
            pltpu.VMEM((L,), jnp.float32),
            pltpu.SemaphoreType.DMA,
        ],
    )
    def k(fm_hbm, lv_hbm, tail_hbm, idx_hbm, fm_out, lin_out, idx_v, buf_v,
          vrow_v, lin_v, tail_v, sem):
        wid = lax.axis_index("s") * NC + lax.axis_index("c")
        base = wid * BPW
        pltpu.sync_copy(idx_hbm.at[pl.ds(base, BPW)], idx_v)
        cp = pltpu.async_copy(fm_hbm.at[idx_v], buf_v, sem)
        pltpu.sync_copy(tail_hbm, tail_v)

        @pl.loop(0, BPW, step=L)
        def _(i):
            c = idx_v[pl.ds(i, L)]
            vrow_v[pl.ds(i, L)] = lax.min(
                lax.shift_right_logical(c, 4),
                jnp.full((L,), LV_ROWS - 1, jnp.int32))

        cp.wait()
        pltpu.sync_copy(buf_v, fm_out.at[pl.ds(base, BPW)])

        pltpu.async_copy(lv_hbm.at[vrow_v], buf_v, sem).wait()

        @pl.loop(0, BPW, step=L)
        def _(i):
            c = idx_v[pl.ds(i, L)]
            lane = lax.bitwise_and(c, L - 1)
            row = lax.iota(jnp.int32, L) + i
            val = plsc.load_gather(buf_v, [row, lane])
            toff = lax.max(c - TAIL0, jnp.zeros((L,), jnp.int32))
            tval = plsc.load_gather(tail_v, [toff])
            lin_v[pl.ds(i, L)] = jnp.where(c >= TAIL0, tval, val)

        pltpu.sync_copy(lin_v, lin_out.at[pl.ds(base, BPW)])

    return k(fm_table, lin_view, lin_tail, idx_flat)


def _afm_body(fm_ref, lin_ref, wbig_ref, hbig_ref, qbig_ref, bias_ref,
              out_ref):
    bt = fm_ref.shape[0]
    fm = fm_ref[...]
    wbig = wbig_ref[...]
    hbig = hbig_ref[...]
    qbig = qbig_ref[...]
    bias = bias_ref[...]

    reps = [jnp.tile(fm[:, i * E:(i + 1) * E], (1, F - 1 - i))
            for i in range(F - 1)]
    rest = [fm[:, (i + 1) * E:] for i in range(F - 1)]
    prod = jnp.concatenate(reps, axis=1) * jnp.concatenate(rest, axis=1)
    prod = jnp.concatenate(
        [prod, jnp.zeros((bt, LPAD), jnp.float32)], axis=1)

    s_parts, q_parts = [], []
    for c in range(NCHUNK):
        pc = prod[:, c * G * E:(c + 1) * G * E]
        zc = jnp.dot(pc, wbig, preferred_element_type=jnp.float32)
        ac = jax.nn.relu(zc + bias)
        s_parts.append(jnp.dot(ac, hbig, preferred_element_type=jnp.float32))
        q_parts.append(jnp.dot(pc, qbig, preferred_element_type=jnp.float32))
    s = jnp.concatenate(s_parts, axis=1)
    q = jnp.concatenate(q_parts, axis=1)

    p_id = jax.lax.broadcasted_iota(jnp.int32, (bt, PPAD), 1)
    mask = p_id < P
    s = jnp.where(mask, s, -1e30)
    m = jnp.max(s, axis=1, keepdims=True)
    e = jnp.where(mask, jnp.exp(s - m), 0.0)
    denom = jnp.sum(e, axis=1, keepdims=True)
    num = jnp.sum(e * q, axis=1, keepdims=True)
    afm = num / denom
    lin_sum = jnp.sum(lin_ref[...], axis=1, keepdims=True)
    out_ref[...] = jax.nn.sigmoid(afm + lin_sum)


def _afm_stage(fm_flat, lin, wbig, hbig, qbig, bias):
    return pl.pallas_call(
        _afm_body,
        grid=(B // BT,),
        in_specs=[
            pl.BlockSpec((BT, F * E), lambda i: (i, 0)),
            pl.BlockSpec((BT, F), lambda i: (i, 0)),
            pl.BlockSpec((G * E, G * 32), lambda i: (0, 0)),
            pl.BlockSpec((G * 32, G), lambda i: (0, 0)),
            pl.BlockSpec((G * E, G), lambda i: (0, 0)),
            pl.BlockSpec((1, G * 32), lambda i: (0, 0)),
        ],
        out_specs=pl.BlockSpec((BT, 1), lambda i: (i, 0)),
        out_shape=jax.ShapeDtypeStruct((B, 1), jnp.float32),
    )(fm_flat, lin, wbig, hbig, qbig, bias)


def kernel(indices, fm_table, linear_table, att_W, att_b, att_h, proj_p,
           training):
    del training
    offsets = (jnp.arange(F, dtype=indices.dtype) * (V + 1))[None, :]
    idx = (indices + offsets).astype(jnp.int32)

    one = jnp.float32(1) - jnp.min(att_b) * 0
    lin_flat = linear_table.reshape(-1)
    lin_view = (lax.slice(lin_flat, (0,), (LV_ROWS * L,)) *
                one).reshape(LV_ROWS, L)
    lin_tail = lax.slice(lin_flat, (TAIL0,), (NLIN,))
    fm_rows, lin_vals = _sc_gather(fm_table, lin_view, lin_tail,
                                   idx.reshape(-1))

    wc = jnp.zeros((E, 32), jnp.float32).at[:, :T].set(att_W)
    eye = jnp.eye(G, dtype=jnp.float32)
    wbig = jnp.kron(eye, wc)
    hpad = jnp.zeros((32, 1), jnp.float32).at[:T, 0].set(att_h[:, 0])
    hbig = jnp.kron(eye, hpad)
    qbig = jnp.kron(eye, proj_p)
    bias = jnp.tile(
        jnp.concatenate([att_b, jnp.zeros((T,), jnp.float32)])[None, :],
        (1, G))

    fm_flat = fm_rows.reshape(B, F * E)
    lin = lin_vals.reshape(B, F)
    return _afm_stage(fm_flat, lin, wbig, hbig, qbig, bias)


# speedup vs baseline: 4.1554x; 1.0014x over previous
import dataclasses
import functools

import jax
import jax.numpy as jnp
from jax import lax
from jax.experimental import pallas as pl
from jax.experimental.pallas import tpu as pltpu
from jax.experimental.pallas import tpu_sc as plsc

F = 26
V = 100000
E = 16
T = 16
B = 4096
P = F * (F - 1) // 2
G = 16
NCHUNK = (P + G - 1) // G
PPAD = NCHUNK * G
LPAD = (PPAD - P) * E

BT = 256

BF = B * F
NC, NS, L = 2, 16, 16
NW = NC * NS
BPW = BF // NW
NLIN = F * (V + 1)
LV_ROWS = NLIN // L
TAIL0 = NLIN - L


def _sc_gather(fm_table, idx_flat):
    mesh = plsc.VectorSubcoreMesh(core_axis_name="c", subcore_axis_name="s")
    cp = pltpu.CompilerParams(
        needs_layout_passes=False, use_tc_tiling_on_sc=False)

    @functools.partial(
        pl.kernel,
        mesh=mesh,
        compiler_params=cp,
        out_type=jax.ShapeDtypeStruct((BF, E), jnp.float32),
        scratch_types=[
            pltpu.VMEM((BPW,), jnp.int32),
            pltpu.VMEM((BPW, E), jnp.float32),
            pltpu.SemaphoreType.DMA,
        ],
    )
    def k(fm_hbm, idx_hbm, fm_out, idx_v, buf_v, sem):
        wid = lax.axis_index("s") * NC + lax.axis_index("c")
        base = wid * BPW
        pltpu.sync_copy(idx_hbm.at[pl.ds(base, BPW)], idx_v)
        pltpu.async_copy(fm_hbm.at[idx_v], buf_v, sem).wait()
        pltpu.sync_copy(buf_v, fm_out.at[pl.ds(base, BPW)])

    return k(fm_table, idx_flat)


def _afm_body(fm_ref, lin_ref, wbig_ref, hbig_ref, qbig_ref, bias_ref,
              out_ref):
    bt = fm_ref.shape[0]
    fm = fm_ref[...]
    wbig = wbig_ref[...]
    hbig = hbig_ref[...]
    qbig = qbig_ref[...]
    bias = bias_ref[...]

    reps = [jnp.tile(fm[:, i * E:(i + 1) * E], (1, F - 1 - i))
            for i in range(F - 1)]
    rest = [fm[:, (i + 1) * E:] for i in range(F - 1)]
    prod = jnp.concatenate(reps, axis=1) * jnp.concatenate(rest, axis=1)
    prod = jnp.concatenate(
        [prod, jnp.zeros((bt, LPAD), jnp.float32)], axis=1)

    s_parts, q_parts = [], []
    for c in range(NCHUNK):
        pc = prod[:, c * G * E:(c + 1) * G * E]
        zc = jnp.dot(pc, wbig, preferred_element_type=jnp.float32)
        ac = jax.nn.relu(zc + bias)
        s_parts.append(jnp.dot(ac, hbig, preferred_element_type=jnp.float32))
        q_parts.append(jnp.dot(pc, qbig, preferred_element_type=jnp.float32))
    s = jnp.concatenate(s_parts, axis=1)
    q = jnp.concatenate(q_parts, axis=1)

    p_id = jax.lax.broadcasted_iota(jnp.int32, (bt, PPAD), 1)
    mask = p_id < P
    s = jnp.where(mask, s, -1e30)
    m = jnp.max(s, axis=1, keepdims=True)
    e = jnp.where(mask, jnp.exp(s - m), 0.0)
    denom = jnp.sum(e, axis=1, keepdims=True)
    num = jnp.sum(e * q, axis=1, keepdims=True)
    afm = num / denom
    lin_sum = jnp.sum(lin_ref[...], axis=1, keepdims=True)
    out_ref[...] = jax.nn.sigmoid(afm + lin_sum)


def _afm_stage(fm_flat, lin, wbig, hbig, qbig, bias):
    return pl.pallas_call(
        _afm_body,
        grid=(B // BT,),
        in_specs=[
            pl.BlockSpec((BT, F * E), lambda i: (i, 0)),
            pl.BlockSpec((BT, F), lambda i: (i, 0)),
            pl.BlockSpec((G * E, G * 32), lambda i: (0, 0)),
            pl.BlockSpec((G * 32, G), lambda i: (0, 0)),
            pl.BlockSpec((G * E, G), lambda i: (0, 0)),
            pl.BlockSpec((1, G * 32), lambda i: (0, 0)),
        ],
        out_specs=pl.BlockSpec((BT, 1), lambda i: (i, 0)),
        out_shape=jax.ShapeDtypeStruct((B, 1), jnp.float32),
    )(fm_flat, lin, wbig, hbig, qbig, bias)


def kernel(indices, fm_table, linear_table, att_W, att_b, att_h, proj_p,
           training):
    del training
    offsets = (jnp.arange(F, dtype=indices.dtype) * (V + 1))[None, :]
    idx = (indices + offsets).astype(jnp.int32)

    fm_rows = _sc_gather(fm_table, idx.reshape(-1))
    lin_vals = jnp.take(linear_table, idx.reshape(-1), axis=0)

    wc = jnp.zeros((E, 32), jnp.float32).at[:, :T].set(att_W)
    eye = jnp.eye(G, dtype=jnp.float32)
    wbig = jnp.kron(eye, wc)
    hpad = jnp.zeros((32, 1), jnp.float32).at[:T, 0].set(att_h[:, 0])
    hbig = jnp.kron(eye, hpad)
    qbig = jnp.kron(eye, proj_p)
    bias = jnp.tile(
        jnp.concatenate([att_b, jnp.zeros((T,), jnp.float32)])[None, :],
        (1, G))

    fm_flat = fm_rows.reshape(B, F * E)
    lin = lin_vals.reshape(B, F)
    return _afm_stage(fm_flat, lin, wbig, hbig, qbig, bias)
